# baseline (speedup 1.0000x reference)
import functools
import math

import jax
import jax.numpy as jnp
from jax.experimental import pallas as pl
from jax.experimental.pallas import tpu as pltpu

SLOT_DIM = 1024
WM_DIM = 1024
NUM_MECH = 16
N_SLOTS = 64
TAU = 1.0

_BB = 128
_SB = 8


def _gelu_exact(x):
    return 0.5 * x * (1.0 + jax.lax.erf(x * (1.0 / math.sqrt(2.0))))


def _body(ns, slots_ref, wm_ref, w1a_ref, w1b_ref, b1_ref, g_ref, beta_ref,
          w2_ref, b2_ref, w3_ref, b3_ref, gn_ref, out_ref, acc_ref):
    j = pl.program_id(1)
    part = jnp.sum(slots_ref[...], axis=1)
    hpart = jnp.dot(part, w1a_ref[...], preferred_element_type=jnp.float32)

    @pl.when(j == 0)
    def _init():
        acc_ref[...] = hpart

    @pl.when(j > 0)
    def _accum():
        acc_ref[...] = acc_ref[...] + hpart

    @pl.when(j == ns - 1)
    def _mlp():
        h = (acc_ref[...]
             + jnp.dot(wm_ref[...], w1b_ref[...], preferred_element_type=jnp.float32)
             + b1_ref[...])
        mu = jnp.mean(h, axis=-1, keepdims=True)
        var = jnp.mean(jnp.square(h - mu), axis=-1, keepdims=True)
        h = (h - mu) * jax.lax.rsqrt(var + 1e-5) * g_ref[...] + beta_ref[...]
        h = _gelu_exact(h)
        h = _gelu_exact(jnp.dot(h, w2_ref[...], preferred_element_type=jnp.float32)
                        + b2_ref[...])
        logits = (jnp.dot(h, w3_ref[...], preferred_element_type=jnp.float32)
                  + b3_ref[...] + gn_ref[...]) * (1.0 / TAU)
        m = jnp.max(logits, axis=-1, keepdims=True)
        e = jnp.exp(logits - m)
        out_ref[...] = e / jnp.sum(e, axis=-1, keepdims=True)


def kernel(slots, working_mem, W1, b1, ln_g, ln_b, W2, b2, W3, b3):
    B = slots.shape[0]
    nb = B // _BB
    ns = N_SLOTS // _SB
    gnoise = jax.random.gumbel(jax.random.key(42), (B, NUM_MECH), dtype=jnp.float32)
    W1a = W1[:SLOT_DIM] * (1.0 / N_SLOTS)
    W1b = W1[SLOT_DIM:]

    grid = (nb, ns)
    return pl.pallas_call(
        functools.partial(_body, ns),
        grid=grid,
        in_specs=[
            pl.BlockSpec((_BB, _SB, SLOT_DIM), lambda i, j: (i, j, 0)),
            pl.BlockSpec((_BB, WM_DIM), lambda i, j: (i, 0)),
            pl.BlockSpec((SLOT_DIM, 512), lambda i, j: (0, 0)),
            pl.BlockSpec((WM_DIM, 512), lambda i, j: (0, 0)),
            pl.BlockSpec((1, 512), lambda i, j: (0, 0)),
            pl.BlockSpec((1, 512), lambda i, j: (0, 0)),
            pl.BlockSpec((1, 512), lambda i, j: (0, 0)),
            pl.BlockSpec((512, 256), lambda i, j: (0, 0)),
            pl.BlockSpec((1, 256), lambda i, j: (0, 0)),
            pl.BlockSpec((256, NUM_MECH), lambda i, j: (0, 0)),
            pl.BlockSpec((1, NUM_MECH), lambda i, j: (0, 0)),
            pl.BlockSpec((_BB, NUM_MECH), lambda i, j: (i, 0)),
        ],
        out_specs=pl.BlockSpec((_BB, NUM_MECH), lambda i, j: (i, 0)),
        out_shape=jax.ShapeDtypeStruct((B, NUM_MECH), jnp.float32),
        scratch_shapes=[pltpu.VMEM((_BB, 512), jnp.float32)],
        compiler_params=pltpu.CompilerParams(
            dimension_semantics=("parallel", "arbitrary"),
        ),
    )(slots, working_mem, W1a, W1b, b1.reshape(1, -1), ln_g.reshape(1, -1),
      ln_b.reshape(1, -1), W2, b2.reshape(1, -1), W3, b3.reshape(1, -1), gnoise)


# speedup vs baseline: 1.2514x; 1.2514x over previous
import math

import jax
import jax.numpy as jnp
from jax.experimental import pallas as pl
from jax.experimental.pallas import tpu as pltpu

SLOT_DIM = 1024
WM_DIM = 1024
NUM_MECH = 16
N_SLOTS = 64
TAU = 1.0

_BB = 32


def _gelu_exact(x):
    return 0.5 * x * (1.0 + jax.lax.erf(x * (1.0 / math.sqrt(2.0))))


def _body(slots_ref, wm_ref, w1_ref, b1_ref, g_ref, beta_ref,
          w2_ref, b2_ref, w3_ref, b3_ref, gn_ref, out_ref):
    t = slots_ref[:, 0:8, :]
    for m in range(1, 8):
        t = t + slots_ref[:, 8 * m:8 * m + 8, :]
    pooled = jnp.sum(t, axis=1) * (1.0 / N_SLOTS)

    h = (jnp.dot(pooled, w1_ref[0:SLOT_DIM, :], preferred_element_type=jnp.float32)
         + jnp.dot(wm_ref[...], w1_ref[SLOT_DIM:, :], preferred_element_type=jnp.float32)
         + b1_ref[...])
    mu = jnp.mean(h, axis=-1, keepdims=True)
    var = jnp.mean(jnp.square(h - mu), axis=-1, keepdims=True)
    h = (h - mu) * jax.lax.rsqrt(var + 1e-5) * g_ref[...] + beta_ref[...]
    h = _gelu_exact(h)
    h = _gelu_exact(jnp.dot(h, w2_ref[...], preferred_element_type=jnp.float32)
                    + b2_ref[...])
    logits = (jnp.dot(h, w3_ref[...], preferred_element_type=jnp.float32)
              + b3_ref[...] + gn_ref[...]) * (1.0 / TAU)
    m = jnp.max(logits, axis=-1, keepdims=True)
    e = jnp.exp(logits - m)
    out_ref[...] = e / jnp.sum(e, axis=-1, keepdims=True)


def kernel(slots, working_mem, W1, b1, ln_g, ln_b, W2, b2, W3, b3):
    B = slots.shape[0]
    nb = B // _BB
    gnoise = jax.random.gumbel(jax.random.key(42), (B, NUM_MECH), dtype=jnp.float32)

    return pl.pallas_call(
        _body,
        grid=(nb,),
        in_specs=[
            pl.BlockSpec((_BB, N_SLOTS, SLOT_DIM), lambda i: (i, 0, 0)),
            pl.BlockSpec((_BB, WM_DIM), lambda i: (i, 0)),
            pl.BlockSpec((SLOT_DIM + WM_DIM, 512), lambda i: (0, 0)),
            pl.BlockSpec((1, 512), lambda i: (0, 0)),
            pl.BlockSpec((1, 512), lambda i: (0, 0)),
            pl.BlockSpec((1, 512), lambda i: (0, 0)),
            pl.BlockSpec((512, 256), lambda i: (0, 0)),
            pl.BlockSpec((1, 256), lambda i: (0, 0)),
            pl.BlockSpec((256, NUM_MECH), lambda i: (0, 0)),
            pl.BlockSpec((1, NUM_MECH), lambda i: (0, 0)),
            pl.BlockSpec((_BB, NUM_MECH), lambda i: (i, 0)),
        ],
        out_specs=pl.BlockSpec((_BB, NUM_MECH), lambda i: (i, 0)),
        out_shape=jax.ShapeDtypeStruct((B, NUM_MECH), jnp.float32),
        compiler_params=pltpu.CompilerParams(
            dimension_semantics=("arbitrary",),
        ),
    )(slots, working_mem, W1, b1.reshape(1, -1), ln_g.reshape(1, -1),
      ln_b.reshape(1, -1), W2, b2.reshape(1, -1), W3, b3.reshape(1, -1), gnoise)


# speedup vs baseline: 1.3317x; 1.0642x over previous
import math

import jax
import jax.numpy as jnp
from jax.experimental import pallas as pl
from jax.experimental.pallas import tpu as pltpu

SLOT_DIM = 1024
WM_DIM = 1024
NUM_MECH = 16
N_SLOTS = 64
TAU = 1.0

_BB = 64


def _gelu_exact(x):
    return 0.5 * x * (1.0 + jax.lax.erf(x * (1.0 / math.sqrt(2.0))))


def _body(slots_ref, wm_ref, w1_ref, b1_ref, g_ref, beta_ref,
          w2_ref, b2_ref, w3_ref, b3_ref, gn_ref, out_ref):
    t = slots_ref[:, 0:8, :]
    for m in range(1, 8):
        t = t + slots_ref[:, 8 * m:8 * m + 8, :]
    pooled = jnp.sum(t, axis=1) * (1.0 / N_SLOTS)

    h = (jnp.dot(pooled, w1_ref[0:SLOT_DIM, :], preferred_element_type=jnp.float32)
         + jnp.dot(wm_ref[...], w1_ref[SLOT_DIM:, :], preferred_element_type=jnp.float32)
         + b1_ref[...])
    mu = jnp.mean(h, axis=-1, keepdims=True)
    var = jnp.mean(jnp.square(h - mu), axis=-1, keepdims=True)
    h = (h - mu) * jax.lax.rsqrt(var + 1e-5) * g_ref[...] + beta_ref[...]
    h = _gelu_exact(h)
    h = _gelu_exact(jnp.dot(h, w2_ref[...], preferred_element_type=jnp.float32)
                    + b2_ref[...])
    logits = (jnp.dot(h, w3_ref[...], preferred_element_type=jnp.float32)
              + b3_ref[...] + gn_ref[...]) * (1.0 / TAU)
    m = jnp.max(logits, axis=-1, keepdims=True)
    e = jnp.exp(logits - m)
    out_ref[...] = e / jnp.sum(e, axis=-1, keepdims=True)


def kernel(slots, working_mem, W1, b1, ln_g, ln_b, W2, b2, W3, b3):
    B = slots.shape[0]
    nb = B // _BB
    gnoise = jax.random.gumbel(jax.random.key(42), (B, NUM_MECH), dtype=jnp.float32)

    return pl.pallas_call(
        _body,
        grid=(nb,),
        in_specs=[
            pl.BlockSpec((_BB, N_SLOTS, SLOT_DIM), lambda i: (i, 0, 0)),
            pl.BlockSpec((_BB, WM_DIM), lambda i: (i, 0)),
            pl.BlockSpec((SLOT_DIM + WM_DIM, 512), lambda i: (0, 0)),
            pl.BlockSpec((1, 512), lambda i: (0, 0)),
            pl.BlockSpec((1, 512), lambda i: (0, 0)),
            pl.BlockSpec((1, 512), lambda i: (0, 0)),
            pl.BlockSpec((512, 256), lambda i: (0, 0)),
            pl.BlockSpec((1, 256), lambda i: (0, 0)),
            pl.BlockSpec((256, NUM_MECH), lambda i: (0, 0)),
            pl.BlockSpec((1, NUM_MECH), lambda i: (0, 0)),
            pl.BlockSpec((_BB, NUM_MECH), lambda i: (i, 0)),
        ],
        out_specs=pl.BlockSpec((_BB, NUM_MECH), lambda i: (i, 0)),
        out_shape=jax.ShapeDtypeStruct((B, NUM_MECH), jnp.float32),
        compiler_params=pltpu.CompilerParams(
            dimension_semantics=("arbitrary",),
        ),
    )(slots, working_mem, W1, b1.reshape(1, -1), ln_g.reshape(1, -1),
      ln_b.reshape(1, -1), W2, b2.reshape(1, -1), W3, b3.reshape(1, -1), gnoise)
